# Initial kernel scaffold; baseline (speedup 1.0000x reference)
import jax
import jax.numpy as jnp
from jax.experimental import pallas as pl


def kernel(x, w, b):
    raise NotImplementedError("write your pallas kernel here")


# speedup vs baseline: 2.9684x; 2.9684x over previous
import jax
import jax.numpy as jnp
from jax.experimental import pallas as pl
from jax.experimental.pallas import tpu as pltpu


def _linear_kernel(x_ref, w_ref, b_ref, o_ref):
    xb = x_ref[...].astype(jnp.bfloat16)
    o_ref[...] = (
        jnp.dot(xb, w_ref[...], preferred_element_type=jnp.float32)
        + b_ref[...]
    )


def kernel(x, w, b):
    m, k = x.shape
    n = w.shape[1]
    tm = 1024

    wb = w.astype(jnp.bfloat16)

    grid = (m // tm,)
    cost = pl.CostEstimate(
        flops=2 * m * k * n,
        transcendentals=0,
        bytes_accessed=4 * m * k + 2 * k * n + 4 * n + 4 * m * n,
    )
    return pl.pallas_call(
        _linear_kernel,
        out_shape=jax.ShapeDtypeStruct((m, n), jnp.float32),
        grid=grid,
        in_specs=[
            pl.BlockSpec((tm, k), lambda i: (i, 0)),
            pl.BlockSpec((k, n), lambda i: (0, 0)),
            pl.BlockSpec((1, n), lambda i: (0, 0)),
        ],
        out_specs=pl.BlockSpec((tm, n), lambda i: (i, 0)),
        compiler_params=pltpu.CompilerParams(
            dimension_semantics=("parallel",),
            vmem_limit_bytes=56 << 20,
        ),
        cost_estimate=cost,
    )(x, wb, b)


# speedup vs baseline: 3.2017x; 1.0786x over previous
import jax
import jax.numpy as jnp
from jax.experimental import pallas as pl
from jax.experimental.pallas import tpu as pltpu


def _linear_kernel(x_ref, w_ref, b_ref, o_ref, wb_ref):
    @pl.when(pl.program_id(0) == 0)
    def _cast_w():
        wb_ref[...] = w_ref[...].astype(jnp.bfloat16)

    xb = x_ref[...].astype(jnp.bfloat16)
    o_ref[...] = (
        jnp.dot(xb, wb_ref[...], preferred_element_type=jnp.float32)
        + b_ref[...]
    )


def kernel(x, w, b):
    m, k = x.shape
    n = w.shape[1]
    tm = 1024

    grid = (m // tm,)
    cost = pl.CostEstimate(
        flops=2 * m * k * n,
        transcendentals=0,
        bytes_accessed=4 * m * k + 4 * k * n + 4 * n + 4 * m * n,
    )
    return pl.pallas_call(
        _linear_kernel,
        out_shape=jax.ShapeDtypeStruct((m, n), jnp.float32),
        grid=grid,
        in_specs=[
            pl.BlockSpec((tm, k), lambda i: (i, 0)),
            pl.BlockSpec((k, n), lambda i: (0, 0)),
            pl.BlockSpec((1, n), lambda i: (0, 0)),
        ],
        out_specs=pl.BlockSpec((tm, n), lambda i: (i, 0)),
        scratch_shapes=[pltpu.VMEM((k, n), jnp.bfloat16)],
        compiler_params=pltpu.CompilerParams(
            dimension_semantics=("arbitrary",),
            vmem_limit_bytes=60 << 20,
        ),
        cost_estimate=cost,
    )(x, w, b)


# speedup vs baseline: 3.2171x; 1.0048x over previous
import jax
import jax.numpy as jnp
from jax.experimental import pallas as pl
from jax.experimental.pallas import tpu as pltpu


def _linear_kernel(x_ref, w_ref, b_ref, o_ref, wb_ref):
    @pl.when(pl.program_id(0) == 0)
    def _cast_w():
        wb_ref[...] = w_ref[...].astype(jnp.bfloat16)

    xb = x_ref[...].astype(jnp.bfloat16)
    o_ref[...] = (
        jnp.dot(xb, wb_ref[...], preferred_element_type=jnp.float32)
        + b_ref[...]
    )


def kernel(x, w, b):
    m, k = x.shape
    n = w.shape[1]
    tm = 512

    grid = (m // tm,)
    cost = pl.CostEstimate(
        flops=2 * m * k * n,
        transcendentals=0,
        bytes_accessed=4 * m * k + 4 * k * n + 4 * n + 4 * m * n,
    )
    return pl.pallas_call(
        _linear_kernel,
        out_shape=jax.ShapeDtypeStruct((m, n), jnp.float32),
        grid=grid,
        in_specs=[
            pl.BlockSpec((tm, k), lambda i: (i, 0)),
            pl.BlockSpec((k, n), lambda i: (0, 0)),
            pl.BlockSpec((1, n), lambda i: (0, 0)),
        ],
        out_specs=pl.BlockSpec((tm, n), lambda i: (i, 0)),
        scratch_shapes=[pltpu.VMEM((k, n), jnp.bfloat16)],
        compiler_params=pltpu.CompilerParams(
            dimension_semantics=("arbitrary",),
            vmem_limit_bytes=60 << 20,
        ),
        cost_estimate=cost,
    )(x, w, b)
